# baseline (speedup 1.0000x reference)
import functools

import jax
import jax.numpy as jnp
from jax import lax
from jax.experimental import pallas as pl
from jax.experimental.pallas import tpu as pltpu
from jax.experimental.pallas import tpu_sc as plsc

N = 10000
NP = 10240
D = 128
E = 320000
NC = 2
NS = 16
NW = NC * NS
EPW = E // NW
CH = 80
NCHUNK = EPW // CH
RPT = NP // NS
CNTW = 16


def _sc_agg_body(ei_hbm, h_hbm, out_hbm,
                 sidx, didx, rows0, rows1, acc_sh, sem0, sem1):
    c = lax.axis_index("c")
    s = lax.axis_index("s")
    wid = s * NC + c

    zeros16 = jnp.zeros((16,), jnp.float32)

    def zrow(i, carry):
        for k in range(D // 16):
            rows0[i, pl.ds(k * 16, 16)] = zeros16
        return carry

    lax.fori_loop(0, CH, zrow, 0)
    zh = [pltpu.async_copy(rows0, acc_sh.at[pl.ds(s * RPT + k * CH, CH)],
                           sem0)
          for k in range(RPT // CH)]
    pltpu.sync_copy(ei_hbm.at[0, pl.ds(wid * EPW, EPW)], sidx)
    pltpu.sync_copy(ei_hbm.at[1, pl.ds(wid * EPW, EPW)], didx)
    for hcopy in zh:
        hcopy.wait()

    plsc.subcore_barrier()

    def sl(r, j):
        return r.at[pl.ds(j * CH, CH)]

    pltpu.async_copy(h_hbm.at[sl(sidx, 0)], rows0, sem0)
    pltpu.async_copy(h_hbm.at[sl(sidx, 1)], rows1, sem1)

    def step(i, carry):
        j = i * 2
        pltpu.make_async_copy(h_hbm.at[sl(sidx, j)], rows0, sem0).wait()
        pltpu.async_copy(h_hbm.at[sl(sidx, j + 2)], rows0, sem0)
        pltpu.sync_copy(rows0, acc_sh.at[sl(didx, j)], add=True)
        pltpu.make_async_copy(h_hbm.at[sl(sidx, j + 1)], rows1, sem1).wait()

        @pl.when(i < (NCHUNK - 1) // 2 - 1)
        def _():
            pltpu.async_copy(h_hbm.at[sl(sidx, j + 3)], rows1, sem1)

        pltpu.sync_copy(rows1, acc_sh.at[sl(didx, j + 1)], add=True)
        return carry

    lax.fori_loop(0, (NCHUNK - 1) // 2, step, 0)
    pltpu.make_async_copy(h_hbm.at[sl(sidx, NCHUNK - 1)], rows0, sem0).wait()
    pltpu.sync_copy(rows0, acc_sh.at[sl(didx, NCHUNK - 1)], add=True)

    plsc.subcore_barrier()

    pltpu.sync_copy(acc_sh.at[pl.ds(s * RPT, RPT)],
                    out_hbm.at[c, pl.ds(s * RPT, RPT)])


@functools.cache
def _make_sc_agg():
    mesh = plsc.VectorSubcoreMesh(core_axis_name="c", subcore_axis_name="s",
                                  num_cores=NC, num_subcores=NS)
    scratch = [
        pltpu.VMEM((EPW,), jnp.int32),
        pltpu.VMEM((EPW,), jnp.int32),
        pltpu.VMEM((CH, D), jnp.float32),
        pltpu.VMEM((CH, D), jnp.float32),
        pltpu.VMEM_SHARED((NP, D), jnp.float32),
        pltpu.SemaphoreType.DMA,
        pltpu.SemaphoreType.DMA,
    ]
    return pl.kernel(
        _sc_agg_body,
        out_type=jax.ShapeDtypeStruct((NC, NP, D), jnp.float32),
        mesh=mesh,
        scratch_types=scratch,
        compiler_params=pltpu.CompilerParams(use_tc_tiling_on_sc=False),
    )


def _sc_count_body(ei_hbm, cnt_hbm, didx, cbuf, cnt_sh, sem0):
    c = lax.axis_index("c")
    s = lax.axis_index("s")
    wid = s * NC + c

    zeros16 = jnp.zeros((16,), jnp.float32)

    def zrow(i, carry):
        cbuf[i, pl.ds(0, CNTW)] = zeros16
        return carry

    lax.fori_loop(0, CH, zrow, 0)
    zh = [pltpu.async_copy(cbuf, cnt_sh.at[pl.ds(s * RPT + k * CH, CH)],
                           sem0)
          for k in range(RPT // CH)]
    pltpu.sync_copy(ei_hbm.at[1, pl.ds(wid * EPW, EPW)], didx)
    for hcopy in zh:
        hcopy.wait()

    ones16 = jnp.ones((16,), jnp.float32)

    def orow(i, carry):
        cbuf[i, pl.ds(0, CNTW)] = ones16
        return carry

    lax.fori_loop(0, CH, orow, 0)

    plsc.subcore_barrier()

    def step(j, carry):
        pltpu.sync_copy(cbuf, cnt_sh.at[didx.at[pl.ds(j * CH, CH)]], add=True)
        return carry

    lax.fori_loop(0, NCHUNK, step, 0)

    plsc.subcore_barrier()
    pltpu.sync_copy(cnt_sh.at[pl.ds(s * RPT, RPT)],
                    cnt_hbm.at[c, pl.ds(s * RPT, RPT)])


@functools.cache
def _make_sc_count():
    mesh = plsc.VectorSubcoreMesh(core_axis_name="c", subcore_axis_name="s",
                                  num_cores=NC, num_subcores=NS)
    scratch = [
        pltpu.VMEM((EPW,), jnp.int32),
        pltpu.VMEM((CH, CNTW), jnp.float32),
        pltpu.VMEM_SHARED((NP, CNTW), jnp.float32),
        pltpu.SemaphoreType.DMA,
    ]
    return pl.kernel(
        _sc_count_body,
        out_type=jax.ShapeDtypeStruct((NC, NP, CNTW), jnp.float32),
        mesh=mesh,
        scratch_types=scratch,
        compiler_params=pltpu.CompilerParams(use_tc_tiling_on_sc=False),
    )


def _dense_body(h_ref, a_ref, c_ref, wl_ref, bl_ref,
                wr_ref, o_ref):
    summed = a_ref[0] + a_ref[1]
    cnt = c_ref[0][:, :1] + c_ref[1][:, :1]
    mean = summed / jnp.maximum(cnt, 1.0)
    out = jnp.dot(mean, wl_ref[...], preferred_element_type=jnp.float32)
    out = out + jnp.dot(h_ref[...], wr_ref[...],
                        preferred_element_type=jnp.float32)
    o_ref[...] = jnp.maximum(out + bl_ref[...], 0.0)


_BN = 5000


def _dense_layer(h, agg, cnt, wlT, bl, wrT):
    grid = (N // _BN,)
    return pl.pallas_call(
        _dense_body,
        grid=grid,
        in_specs=[
            pl.BlockSpec((_BN, D), lambda i: (i, 0)),
            pl.BlockSpec((NC, _BN, D), lambda i: (0, i, 0)),
            pl.BlockSpec((NC, _BN, CNTW), lambda i: (0, i, 0)),
            pl.BlockSpec((D, D), lambda i: (0, 0)),
            pl.BlockSpec((1, D), lambda i: (0, 0)),
            pl.BlockSpec((D, D), lambda i: (0, 0)),
        ],
        out_specs=pl.BlockSpec((_BN, D), lambda i: (i, 0)),
        out_shape=jax.ShapeDtypeStruct((N, D), jnp.float32),
    )(h, agg, cnt, wlT, bl, wrT)


def _dense_mlp_body(h1_ref, a_ref, c_ref, wl_ref, bl_ref, wr_ref,
                    h0_ref, w2a_ref, w2b_ref, w2c_ref, b2_ref,
                    w3_ref, b3_ref, o_ref):
    summed = a_ref[0] + a_ref[1]
    cnt = c_ref[0][:, :1] + c_ref[1][:, :1]
    mean = summed / jnp.maximum(cnt, 1.0)
    out = jnp.dot(mean, wl_ref[...], preferred_element_type=jnp.float32)
    out = out + jnp.dot(h1_ref[...], wr_ref[...],
                        preferred_element_type=jnp.float32)
    h2 = jnp.maximum(out + bl_ref[...], 0.0)
    p = jnp.dot(h0_ref[...], w2a_ref[...], preferred_element_type=jnp.float32)
    p = p + jnp.dot(h1_ref[...], w2b_ref[...],
                    preferred_element_type=jnp.float32)
    p = p + jnp.dot(h2, w2c_ref[...], preferred_element_type=jnp.float32)
    p = jnp.maximum(p + b2_ref[...], 0.0)
    o_ref[...] = jnp.dot(p, w3_ref[...],
                         preferred_element_type=jnp.float32) + b3_ref[...]


def _dense_mlp(h1, agg, cnt, wlT, bl, wrT, h0, w2a, w2b, w2c, b2, w3T, b3):
    D2 = 2 * D
    grid = (N // _BN,)
    return pl.pallas_call(
        _dense_mlp_body,
        grid=grid,
        in_specs=[
            pl.BlockSpec((_BN, D), lambda i: (i, 0)),
            pl.BlockSpec((NC, _BN, D), lambda i: (0, i, 0)),
            pl.BlockSpec((NC, _BN, CNTW), lambda i: (0, i, 0)),
            pl.BlockSpec((D, D), lambda i: (0, 0)),
            pl.BlockSpec((1, D), lambda i: (0, 0)),
            pl.BlockSpec((D, D), lambda i: (0, 0)),
            pl.BlockSpec((_BN, D), lambda i: (i, 0)),
            pl.BlockSpec((D, D2), lambda i: (0, 0)),
            pl.BlockSpec((D, D2), lambda i: (0, 0)),
            pl.BlockSpec((D, D2), lambda i: (0, 0)),
            pl.BlockSpec((1, D2), lambda i: (0, 0)),
            pl.BlockSpec((D2, D), lambda i: (0, 0)),
            pl.BlockSpec((1, D), lambda i: (0, 0)),
        ],
        out_specs=pl.BlockSpec((_BN, D), lambda i: (i, 0)),
        out_shape=jax.ShapeDtypeStruct((N, D), jnp.float32),
    )(h1, agg, cnt, wlT, bl, wrT, h0, w2a, w2b, w2c, b2, w3T, b3)


def kernel(x, edge_index, Wl0, bl0, Wr0, Wl1, bl1, Wr1, Wl2, bl2, Wr2,
           W2, b2, W3, b3):
    ei = edge_index.astype(jnp.int32)

    wlT = [Wl0.T, Wl1.T, Wl2.T]
    wrT = [Wr0.T, Wr1.T, Wr2.T]
    bl = [bl0.reshape(1, D), bl1.reshape(1, D), bl2.reshape(1, D)]

    cnt = _make_sc_count()(ei)
    h0 = _dense_layer(x, _make_sc_agg()(ei, x), cnt, wlT[0], bl[0], wrT[0])
    h1 = _dense_layer(h0, _make_sc_agg()(ei, h0), cnt, wlT[1], bl[1], wrT[1])
    agg2 = _make_sc_agg()(ei, h1)
    w2T = W2.T
    out = _dense_mlp(h1, agg2, cnt, wlT[2], bl[2], wrT[2], h0,
                     w2T[0:D], w2T[D:2 * D], w2T[2 * D:3 * D],
                     b2.reshape(1, 2 * D), W3.T, b3.reshape(1, D))
    return out


# speedup vs baseline: 1.0048x; 1.0048x over previous
import functools

import jax
import jax.numpy as jnp
from jax import lax
from jax.experimental import pallas as pl
from jax.experimental.pallas import tpu as pltpu
from jax.experimental.pallas import tpu_sc as plsc

N = 10000
NP = 10240
D = 128
E = 320000
NC = 2
NS = 16
NW = NC * NS
EPW = E // NW
CH = 80
NCHUNK = EPW // CH
RPT = NP // NS
CNTW = 16


def _sc_agg_body(ei_hbm, h_hbm, out_hbm,
                 sidx, didx, rows0, rows1, acc_sh, sem0, sem1):
    c = lax.axis_index("c")
    s = lax.axis_index("s")
    wid = s * NC + c

    zeros16 = jnp.zeros((16,), jnp.float32)

    def zrow(i, carry):
        for k in range(D // 16):
            rows0[i, pl.ds(k * 16, 16)] = zeros16
        return carry

    lax.fori_loop(0, CH, zrow, 0)
    zh = [pltpu.async_copy(rows0, acc_sh.at[pl.ds(s * RPT + k * CH, CH)],
                           sem0)
          for k in range(RPT // CH)]
    pltpu.sync_copy(ei_hbm.at[0, pl.ds(wid * EPW, EPW)], sidx)
    pltpu.sync_copy(ei_hbm.at[1, pl.ds(wid * EPW, EPW)], didx)
    for hcopy in zh:
        hcopy.wait()

    plsc.subcore_barrier()

    def sl(r, j):
        return r.at[pl.ds(j * CH, CH)]

    pltpu.async_copy(h_hbm.at[sl(sidx, 0)], rows0, sem0)
    pltpu.async_copy(h_hbm.at[sl(sidx, 1)], rows1, sem1)

    def step(i, carry):
        j = i * 2
        pltpu.make_async_copy(h_hbm.at[sl(sidx, j)], rows0, sem0).wait()
        pltpu.async_copy(h_hbm.at[sl(sidx, j + 2)], rows0, sem0)
        pltpu.sync_copy(rows0, acc_sh.at[sl(didx, j)], add=True)
        pltpu.make_async_copy(h_hbm.at[sl(sidx, j + 1)], rows1, sem1).wait()

        @pl.when(i < (NCHUNK - 1) // 2 - 1)
        def _():
            pltpu.async_copy(h_hbm.at[sl(sidx, j + 3)], rows1, sem1)

        pltpu.sync_copy(rows1, acc_sh.at[sl(didx, j + 1)], add=True)
        return carry

    lax.fori_loop(0, (NCHUNK - 1) // 2, step, 0)
    pltpu.make_async_copy(h_hbm.at[sl(sidx, NCHUNK - 1)], rows0, sem0).wait()
    pltpu.sync_copy(rows0, acc_sh.at[sl(didx, NCHUNK - 1)], add=True)

    plsc.subcore_barrier()

    pltpu.sync_copy(acc_sh.at[pl.ds(s * RPT, RPT)],
                    out_hbm.at[c, pl.ds(s * RPT, RPT)])


@functools.cache
def _make_sc_agg():
    mesh = plsc.VectorSubcoreMesh(core_axis_name="c", subcore_axis_name="s",
                                  num_cores=NC, num_subcores=NS)
    scratch = [
        pltpu.VMEM((EPW,), jnp.int32),
        pltpu.VMEM((EPW,), jnp.int32),
        pltpu.VMEM((CH, D), jnp.float32),
        pltpu.VMEM((CH, D), jnp.float32),
        pltpu.VMEM_SHARED((NP, D), jnp.float32),
        pltpu.SemaphoreType.DMA,
        pltpu.SemaphoreType.DMA,
    ]
    return pl.kernel(
        _sc_agg_body,
        out_type=jax.ShapeDtypeStruct((NC, NP, D), jnp.float32),
        mesh=mesh,
        scratch_types=scratch,
        compiler_params=pltpu.CompilerParams(use_tc_tiling_on_sc=False),
    )


def _sc_count_body(ei_hbm, cnt_hbm, didx, cbuf, cnt_sh, sem0):
    c = lax.axis_index("c")
    s = lax.axis_index("s")
    wid = s * NC + c

    zeros16 = jnp.zeros((16,), jnp.float32)

    def zrow(i, carry):
        cbuf[i, pl.ds(0, CNTW)] = zeros16
        return carry

    lax.fori_loop(0, CH, zrow, 0)
    zh = [pltpu.async_copy(cbuf, cnt_sh.at[pl.ds(s * RPT + k * CH, CH)],
                           sem0)
          for k in range(RPT // CH)]
    pltpu.sync_copy(ei_hbm.at[1, pl.ds(wid * EPW, EPW)], didx)
    for hcopy in zh:
        hcopy.wait()

    ones16 = jnp.ones((16,), jnp.float32)

    def orow(i, carry):
        cbuf[i, pl.ds(0, CNTW)] = ones16
        return carry

    lax.fori_loop(0, CH, orow, 0)

    plsc.subcore_barrier()

    def step(j, carry):
        pltpu.sync_copy(cbuf, cnt_sh.at[didx.at[pl.ds(j * CH, CH)]], add=True)
        return carry

    lax.fori_loop(0, NCHUNK, step, 0)

    plsc.subcore_barrier()
    pltpu.sync_copy(cnt_sh.at[pl.ds(s * RPT, RPT)],
                    cnt_hbm.at[c, pl.ds(s * RPT, RPT)])


@functools.cache
def _make_sc_count():
    mesh = plsc.VectorSubcoreMesh(core_axis_name="c", subcore_axis_name="s",
                                  num_cores=NC, num_subcores=NS)
    scratch = [
        pltpu.VMEM((EPW,), jnp.int32),
        pltpu.VMEM((CH, CNTW), jnp.float32),
        pltpu.VMEM_SHARED((NP, CNTW), jnp.float32),
        pltpu.SemaphoreType.DMA,
    ]
    return pl.kernel(
        _sc_count_body,
        out_type=jax.ShapeDtypeStruct((NC, NP, CNTW), jnp.float32),
        mesh=mesh,
        scratch_types=scratch,
        compiler_params=pltpu.CompilerParams(use_tc_tiling_on_sc=False),
    )


def _dense_body(h_ref, a_ref, c_ref, wl_ref, bl_ref,
                wr_ref, o_ref):
    summed = a_ref[0] + a_ref[1]
    cnt = c_ref[0][:, :1] + c_ref[1][:, :1]
    mean = summed / jnp.maximum(cnt, 1.0)
    out = jnp.dot(mean, wl_ref[...], preferred_element_type=jnp.float32)
    out = out + jnp.dot(h_ref[...], wr_ref[...],
                        preferred_element_type=jnp.float32)
    o_ref[...] = jnp.maximum(out + bl_ref[...], 0.0)


_BN = 2000


def _dense_layer(h, agg, cnt, wlT, bl, wrT):
    grid = (N // _BN,)
    return pl.pallas_call(
        _dense_body,
        grid=grid,
        in_specs=[
            pl.BlockSpec((_BN, D), lambda i: (i, 0)),
            pl.BlockSpec((NC, _BN, D), lambda i: (0, i, 0)),
            pl.BlockSpec((NC, _BN, CNTW), lambda i: (0, i, 0)),
            pl.BlockSpec((D, D), lambda i: (0, 0)),
            pl.BlockSpec((1, D), lambda i: (0, 0)),
            pl.BlockSpec((D, D), lambda i: (0, 0)),
        ],
        out_specs=pl.BlockSpec((_BN, D), lambda i: (i, 0)),
        out_shape=jax.ShapeDtypeStruct((N, D), jnp.float32),
    )(h, agg, cnt, wlT, bl, wrT)


def _dense_mlp_body(h1_ref, a_ref, c_ref, wl_ref, bl_ref, wr_ref,
                    h0_ref, w2a_ref, w2b_ref, w2c_ref, b2_ref,
                    w3_ref, b3_ref, o_ref):
    summed = a_ref[0] + a_ref[1]
    cnt = c_ref[0][:, :1] + c_ref[1][:, :1]
    mean = summed / jnp.maximum(cnt, 1.0)
    out = jnp.dot(mean, wl_ref[...], preferred_element_type=jnp.float32)
    out = out + jnp.dot(h1_ref[...], wr_ref[...],
                        preferred_element_type=jnp.float32)
    h2 = jnp.maximum(out + bl_ref[...], 0.0)
    p = jnp.dot(h0_ref[...], w2a_ref[...], preferred_element_type=jnp.float32)
    p = p + jnp.dot(h1_ref[...], w2b_ref[...],
                    preferred_element_type=jnp.float32)
    p = p + jnp.dot(h2, w2c_ref[...], preferred_element_type=jnp.float32)
    p = jnp.maximum(p + b2_ref[...], 0.0)
    o_ref[...] = jnp.dot(p, w3_ref[...],
                         preferred_element_type=jnp.float32) + b3_ref[...]


def _dense_mlp(h1, agg, cnt, wlT, bl, wrT, h0, w2a, w2b, w2c, b2, w3T, b3):
    D2 = 2 * D
    grid = (N // _BN,)
    return pl.pallas_call(
        _dense_mlp_body,
        grid=grid,
        in_specs=[
            pl.BlockSpec((_BN, D), lambda i: (i, 0)),
            pl.BlockSpec((NC, _BN, D), lambda i: (0, i, 0)),
            pl.BlockSpec((NC, _BN, CNTW), lambda i: (0, i, 0)),
            pl.BlockSpec((D, D), lambda i: (0, 0)),
            pl.BlockSpec((1, D), lambda i: (0, 0)),
            pl.BlockSpec((D, D), lambda i: (0, 0)),
            pl.BlockSpec((_BN, D), lambda i: (i, 0)),
            pl.BlockSpec((D, D2), lambda i: (0, 0)),
            pl.BlockSpec((D, D2), lambda i: (0, 0)),
            pl.BlockSpec((D, D2), lambda i: (0, 0)),
            pl.BlockSpec((1, D2), lambda i: (0, 0)),
            pl.BlockSpec((D2, D), lambda i: (0, 0)),
            pl.BlockSpec((1, D), lambda i: (0, 0)),
        ],
        out_specs=pl.BlockSpec((_BN, D), lambda i: (i, 0)),
        out_shape=jax.ShapeDtypeStruct((N, D), jnp.float32),
    )(h1, agg, cnt, wlT, bl, wrT, h0, w2a, w2b, w2c, b2, w3T, b3)


def kernel(x, edge_index, Wl0, bl0, Wr0, Wl1, bl1, Wr1, Wl2, bl2, Wr2,
           W2, b2, W3, b3):
    ei = edge_index.astype(jnp.int32)

    wlT = [Wl0.T, Wl1.T, Wl2.T]
    wrT = [Wr0.T, Wr1.T, Wr2.T]
    bl = [bl0.reshape(1, D), bl1.reshape(1, D), bl2.reshape(1, D)]

    cnt = _make_sc_count()(ei)
    h0 = _dense_layer(x, _make_sc_agg()(ei, x), cnt, wlT[0], bl[0], wrT[0])
    h1 = _dense_layer(h0, _make_sc_agg()(ei, h0), cnt, wlT[1], bl[1], wrT[1])
    agg2 = _make_sc_agg()(ei, h1)
    w2T = W2.T
    out = _dense_mlp(h1, agg2, cnt, wlT[2], bl[2], wrT[2], h0,
                     w2T[0:D], w2T[D:2 * D], w2T[2 * D:3 * D],
                     b2.reshape(1, 2 * D), W3.T, b3.reshape(1, D))
    return out
